# baseline (speedup 1.0000x reference)
import functools

import jax
import jax.numpy as jnp
from jax import lax
from jax.experimental import pallas as pl
from jax.experimental.pallas import tpu as pltpu
from jax.experimental.pallas import tpu_sc as plsc

N = 10000
E = 320000
DIN = 128
H = 64
NG = 16

NC = 2
NS = 16
NW = NC * NS
EPW = E // NW
CH = 80
NCH = EPW // CH
ROWS_PT = 632
NPAD = NS * ROWS_PT
SROWS = 632
FH = NPAD // 2
BRF = 632
NBLKF = FH // BRF


def _sc_segsum_body(vals_hbm, edges_hbm, z_hbm, out_hbm,
                    sidx, didx, rows_a, rows_b, table, acc, sem_a, sem_b):
    cid = lax.axis_index("c")
    sid = lax.axis_index("s")
    wid = cid * NS + sid

    pltpu.sync_copy(edges_hbm.at[0, wid], sidx)
    pltpu.sync_copy(edges_hbm.at[1, wid], didx)
    pltpu.sync_copy(vals_hbm.at[pl.ds(sid * SROWS, SROWS)],
                    table.at[pl.ds(sid * SROWS, SROWS)])
    pltpu.sync_copy(z_hbm, acc.at[pl.ds(sid * ROWS_PT, ROWS_PT)])

    def gather(j, buf, sem):
        pltpu.async_copy(table.at[sidx.at[j]], buf, sem)

    def gather_h(j, buf, sem):
        pltpu.async_copy(vals_hbm.at[sidx.at[j]], buf, sem)

    def wait(buf, sem):
        pltpu.make_async_copy(vals_hbm.at[pl.ds(0, CH)], buf, sem).wait()

    def scatter(j, buf):
        pltpu.sync_copy(buf, acc.at[didx.at[j]], add=True)

    plsc.subcore_barrier()
    gather(0, rows_a, sem_a)

    def pair(jp, carry):
        a = 2 * jp
        gather_h(a + 1, rows_b, sem_b)
        wait(rows_a, sem_a)
        scatter(a, rows_a)
        gather(a + 2, rows_a, sem_a)
        wait(rows_b, sem_b)
        scatter(a + 1, rows_b)
        return carry

    lax.fori_loop(0, (NCH - 1) // 2, pair, 0)
    wait(rows_a, sem_a)
    scatter(NCH - 1, rows_a)

    plsc.subcore_barrier()
    pltpu.sync_copy(acc.at[pl.ds(sid * ROWS_PT, ROWS_PT)],
                    out_hbm.at[pl.ds(cid * NPAD + sid * ROWS_PT, ROWS_PT)])


_sc_mesh = plsc.VectorSubcoreMesh(core_axis_name="c", subcore_axis_name="s",
                                  num_cores=NC, num_subcores=NS)

_sc_segsum = functools.partial(
    pl.kernel,
    out_type=jax.ShapeDtypeStruct((NC * NPAD, H), jnp.float32),
    mesh=_sc_mesh,
    scratch_types=[
        pltpu.VMEM((NCH, CH), jnp.int32),
        pltpu.VMEM((NCH, CH), jnp.int32),
        pltpu.VMEM((CH, H), jnp.float32),
        pltpu.VMEM((CH, H), jnp.float32),
        pltpu.VMEM_SHARED((NPAD, H), jnp.float32),
        pltpu.VMEM_SHARED((NPAD, H), jnp.float32),
        pltpu.SemaphoreType.DMA,
        pltpu.SemaphoreType.DMA,
    ],
    compiler_params=pltpu.CompilerParams(use_tc_tiling_on_sc=False),
)(_sc_segsum_body)


def _dot(a, b):
    return jnp.dot(a, b, preferred_element_type=jnp.float32)


def _tc_layer1_body(xa_ref, xb_ref, wall, ball, base_ref, c_ref):
    xcat = jnp.concatenate([xa_ref[...], xb_ref[...]], axis=1)
    p = _dot(xcat, wall[...]) + ball[...]
    F = 2 * H
    base_ref[...] = p[:, :F] + p[:, F:2 * F] * p[:, 2 * F:3 * F]
    c_ref[...] = p[:, 3 * F:]


def _tc_mid_body(base_ref, g_ref, bc, wall, ball, baseo_ref, co_ref):
    h = jnp.maximum(base_ref[...] + g_ref[0] + g_ref[1] + bc[...], 0.0)
    p = _dot(h, wall[...]) + ball[...]
    F = 2 * H
    baseo_ref[...] = p[:, :F] + p[:, F:2 * F] * p[:, 2 * F:3 * F]
    co_ref[...] = p[:, 3 * F:]


def _tc_final_body(base_ref, g_ref, bc, oht_ref, ohb_ref,
                   wf1, bf1, wf2, bf2, out_ref, acc, cnt):
    i = pl.program_id(0)

    @pl.when(i == 0)
    def _():
        acc[...] = jnp.zeros_like(acc)
        cnt[...] = jnp.zeros_like(cnt)

    h = jnp.maximum(base_ref[...] + g_ref[0] + g_ref[1] + bc[...], 0.0)
    seg = lax.broadcasted_iota(jnp.int32, (1, NG), 1)
    oht = (oht_ref[...] == seg).astype(jnp.float32)
    ohb = (ohb_ref[...] == seg).astype(jnp.float32)

    def _dgt(oh, v):
        return lax.dot_general(oh, v, (((0,), (0,)), ((), ())),
                               preferred_element_type=jnp.float32)

    ones = jnp.ones((BRF, H), jnp.float32)
    acc[...] += _dgt(oht, h[:, :H]) + _dgt(ohb, h[:, H:])
    cnt[...] += _dgt(oht, ones) + _dgt(ohb, ones)

    @pl.when(i == NBLKF - 1)
    def _():
        pooled = acc[...] / jnp.maximum(cnt[...], 1.0)
        t = _dot(pooled, wf1[...]) + bf1[...]
        out_ref[...] = _dot(t, wf2[...]) + bf2[...]


def _full(shape):
    return pl.BlockSpec(shape, lambda i: tuple(0 for _ in shape))


def _frows():
    return pl.BlockSpec((BRF, 2 * H), lambda i: (i, 0))


def _tc_layer1(xpad, wall, ball):
    xa = pl.BlockSpec((BRF, DIN), lambda i: (i, 0))
    xb = pl.BlockSpec((BRF, DIN), lambda i: (i + NBLKF, 0))
    return pl.pallas_call(
        _tc_layer1_body,
        grid=(NBLKF,),
        in_specs=[xa, xb, _full((2 * DIN, 8 * H)), _full((1, 8 * H))],
        out_specs=[_frows(), _frows()],
        out_shape=[jax.ShapeDtypeStruct((FH, 2 * H), jnp.float32),
                   jax.ShapeDtypeStruct((FH, 2 * H), jnp.float32)],
        compiler_params=pltpu.CompilerParams(
            dimension_semantics=("parallel",)),
    )(xpad, xpad, wall, ball)


def _tc_mid(base, g, bc, wall, ball):
    gspec = pl.BlockSpec((NC, BRF, 2 * H), lambda i: (0, i, 0))
    return pl.pallas_call(
        _tc_mid_body,
        grid=(NBLKF,),
        in_specs=[_frows(), gspec, _full((1, 2 * H)),
                  _full((2 * H, 8 * H)), _full((1, 8 * H))],
        out_specs=[_frows(), _frows()],
        out_shape=[jax.ShapeDtypeStruct((FH, 2 * H), jnp.float32),
                   jax.ShapeDtypeStruct((FH, 2 * H), jnp.float32)],
        compiler_params=pltpu.CompilerParams(
            dimension_semantics=("parallel",)),
    )(base, g, bc, wall, ball)


def _tc_final(base, g, bc, onehot, wf1, bf1, wf2, bf2, ncls):
    gspec = pl.BlockSpec((NC, BRF, 2 * H), lambda i: (0, i, 0))
    oht = pl.BlockSpec((BRF, 1), lambda i: (i, 0))
    ohb = pl.BlockSpec((BRF, 1), lambda i: (i + NBLKF, 0))
    return pl.pallas_call(
        _tc_final_body,
        grid=(NBLKF,),
        in_specs=[_frows(), gspec, _full((1, 2 * H)), oht, ohb,
                  _full((H, 10)), _full((1, 10)),
                  _full((10, ncls)), _full((1, ncls))],
        out_specs=_full((NG, ncls)),
        out_shape=jax.ShapeDtypeStruct((NG, ncls), jnp.float32),
        scratch_shapes=[pltpu.VMEM((NG, H), jnp.float32),
                        pltpu.VMEM((NG, H), jnp.float32)],
        compiler_params=pltpu.CompilerParams(
            dimension_semantics=("arbitrary",)),
    )(base, g, bc, onehot, onehot, wf1, bf1, wf2, bf2)


def _fold_w(w):
    z = jnp.zeros_like(w)
    return jnp.concatenate([jnp.concatenate([w, z], axis=1),
                            jnp.concatenate([z, w], axis=1)], axis=0)


def _fold_b(b):
    return jnp.concatenate([b, b]).reshape(1, 2 * H)


def kernel(x, edge_index, batch,
           W_fc11, b_fc11, W_fc12, b_fc12, W_fc13, b_fc13, W_conv1, b_conv1,
           W_fc21, b_fc21, W_fc22, b_fc22, W_fc23, b_fc23, W_conv2, b_conv2,
           W_fc31, b_fc31, W_fc32, b_fc32, W_fc33, b_fc33, W_conv3, b_conv3,
           W_fc1, b_fc1, W_fc2, b_fc2):
    ncls = W_fc2.shape[1]

    xpad = jnp.concatenate(
        [x, jnp.zeros((NPAD - N, DIN), jnp.float32)], axis=0)

    edges4 = jnp.where(edge_index < FH, 2 * edge_index,
                       2 * (edge_index - FH) + 1
                       ).astype(jnp.int32).reshape(2, NW, NCH, CH)
    zeros = jnp.zeros((ROWS_PT, H), jnp.float32)

    bpad = jnp.concatenate(
        [batch, jnp.full((NPAD - N,), -1, batch.dtype)]
    ).astype(jnp.int32).reshape(NPAD, 1)

    def segsum(c_f):
        out = _sc_segsum(c_f.reshape(NPAD, H), edges4, zeros)
        return out.reshape(NC, FH, 2 * H)

    def wball(w1, b1, w2, b2, w3, b3, wc):
        wall = jnp.concatenate(
            [_fold_w(w1), _fold_w(w2), _fold_w(w3), _fold_w(wc)], axis=1)
        ball = jnp.concatenate(
            [_fold_b(b1), _fold_b(b2), _fold_b(b3),
             jnp.zeros((1, 2 * H), jnp.float32)], axis=1)
        return wall, ball

    base1, c1 = _tc_layer1(
        xpad, *wball(W_fc11, b_fc11, W_fc12, b_fc12, W_fc13, b_fc13,
                     W_conv1))
    g1 = segsum(c1)
    base2, c2 = _tc_mid(
        base1, g1, _fold_b(b_conv1),
        *wball(W_fc21, b_fc21, W_fc22, b_fc22, W_fc23, b_fc23, W_conv2))
    g2 = segsum(c2)
    base3, c3 = _tc_mid(
        base2, g2, _fold_b(b_conv2),
        *wball(W_fc31, b_fc31, W_fc32, b_fc32, W_fc33, b_fc33, W_conv3))
    g3 = segsum(c3)
    return _tc_final(base3, g3, _fold_b(b_conv3), bpad,
                     W_fc1, b_fc1.reshape(1, 10), W_fc2,
                     b_fc2.reshape(1, ncls), ncls)


# speedup vs baseline: 1.1427x; 1.1427x over previous
import functools

import jax
import jax.numpy as jnp
from jax import lax
from jax.experimental import pallas as pl
from jax.experimental.pallas import tpu as pltpu
from jax.experimental.pallas import tpu_sc as plsc

N = 10000
E = 320000
DIN = 128
H = 64
NG = 16

NC = 2
NS = 16
NW = NC * NS
EPW = E // NW
CH = 80
NCH = EPW // CH
ROWS_PT = 632
NPAD = NS * ROWS_PT
SROWS = 632
FH = NPAD // 2
BRF = 1264
NBLKF = FH // BRF


def _sc_segsum_body(vals_hbm, edges_hbm, z_hbm, out_hbm,
                    sidx, didx, rows_a, rows_b, table, acc, sem_a, sem_b):
    cid = lax.axis_index("c")
    sid = lax.axis_index("s")
    wid = cid * NS + sid

    pltpu.sync_copy(edges_hbm.at[0, wid], sidx)
    pltpu.sync_copy(edges_hbm.at[1, wid], didx)
    pltpu.sync_copy(vals_hbm.at[pl.ds(sid * SROWS, SROWS)],
                    table.at[pl.ds(sid * SROWS, SROWS)])
    pltpu.sync_copy(z_hbm, acc.at[pl.ds(sid * ROWS_PT, ROWS_PT)])

    def gather(j, buf, sem):
        pltpu.async_copy(table.at[sidx.at[j]], buf, sem)

    def wait(buf, sem):
        pltpu.make_async_copy(vals_hbm.at[pl.ds(0, CH)], buf, sem).wait()

    def scatter(j, buf):
        pltpu.sync_copy(buf, acc.at[didx.at[j]], add=True)

    plsc.subcore_barrier()
    gather(0, rows_a, sem_a)

    def pair(jp, carry):
        a = 2 * jp
        gather(a + 1, rows_b, sem_b)
        wait(rows_a, sem_a)
        scatter(a, rows_a)
        gather(a + 2, rows_a, sem_a)
        wait(rows_b, sem_b)
        scatter(a + 1, rows_b)
        return carry

    lax.fori_loop(0, (NCH - 1) // 2, pair, 0)
    wait(rows_a, sem_a)
    scatter(NCH - 1, rows_a)

    plsc.subcore_barrier()
    pltpu.sync_copy(acc.at[pl.ds(sid * ROWS_PT, ROWS_PT)],
                    out_hbm.at[pl.ds(cid * NPAD + sid * ROWS_PT, ROWS_PT)])


_sc_mesh = plsc.VectorSubcoreMesh(core_axis_name="c", subcore_axis_name="s",
                                  num_cores=NC, num_subcores=NS)

_sc_segsum = functools.partial(
    pl.kernel,
    out_type=jax.ShapeDtypeStruct((NC * NPAD, H), jnp.float32),
    mesh=_sc_mesh,
    scratch_types=[
        pltpu.VMEM((NCH, CH), jnp.int32),
        pltpu.VMEM((NCH, CH), jnp.int32),
        pltpu.VMEM((CH, H), jnp.float32),
        pltpu.VMEM((CH, H), jnp.float32),
        pltpu.VMEM_SHARED((NPAD, H), jnp.float32),
        pltpu.VMEM_SHARED((NPAD, H), jnp.float32),
        pltpu.SemaphoreType.DMA,
        pltpu.SemaphoreType.DMA,
    ],
    compiler_params=pltpu.CompilerParams(use_tc_tiling_on_sc=False),
)(_sc_segsum_body)


def _dot(a, b):
    return jnp.dot(a, b, preferred_element_type=jnp.float32)


def _tc_layer1_body(xa_ref, xb_ref, wall, ball, base_ref, c_ref):
    xcat = jnp.concatenate([xa_ref[...], xb_ref[...]], axis=1)
    p = _dot(xcat, wall[...]) + ball[...]
    F = 2 * H
    base_ref[...] = p[:, :F] + p[:, F:2 * F] * p[:, 2 * F:3 * F]
    c_ref[...] = p[:, 3 * F:]


def _tc_mid_body(base_ref, g_ref, bc, wall, ball, baseo_ref, co_ref):
    h = jnp.maximum(base_ref[...] + g_ref[0] + g_ref[1] + bc[...], 0.0)
    p = _dot(h, wall[...]) + ball[...]
    F = 2 * H
    baseo_ref[...] = p[:, :F] + p[:, F:2 * F] * p[:, 2 * F:3 * F]
    co_ref[...] = p[:, 3 * F:]


def _tc_final_body(base_ref, g_ref, bc, oht_ref, ohb_ref,
                   wf1, bf1, wf2, bf2, out_ref, acc, cnt):
    i = pl.program_id(0)

    @pl.when(i == 0)
    def _():
        acc[...] = jnp.zeros_like(acc)
        cnt[...] = jnp.zeros_like(cnt)

    h = jnp.maximum(base_ref[...] + g_ref[0] + g_ref[1] + bc[...], 0.0)
    seg = lax.broadcasted_iota(jnp.int32, (1, NG), 1)
    oht = (oht_ref[...] == seg).astype(jnp.float32)
    ohb = (ohb_ref[...] == seg).astype(jnp.float32)

    def _dgt(oh, v):
        return lax.dot_general(oh, v, (((0,), (0,)), ((), ())),
                               preferred_element_type=jnp.float32)

    ones = jnp.ones((BRF, H), jnp.float32)
    acc[...] += _dgt(oht, h[:, :H]) + _dgt(ohb, h[:, H:])
    cnt[...] += _dgt(oht, ones) + _dgt(ohb, ones)

    @pl.when(i == NBLKF - 1)
    def _():
        pooled = acc[...] / jnp.maximum(cnt[...], 1.0)
        t = _dot(pooled, wf1[...]) + bf1[...]
        out_ref[...] = _dot(t, wf2[...]) + bf2[...]


def _full(shape):
    return pl.BlockSpec(shape, lambda i: tuple(0 for _ in shape))


def _frows():
    return pl.BlockSpec((BRF, 2 * H), lambda i: (i, 0))


def _tc_layer1(xpad, wall, ball):
    xa = pl.BlockSpec((BRF, DIN), lambda i: (i, 0))
    xb = pl.BlockSpec((BRF, DIN), lambda i: (i + NBLKF, 0))
    return pl.pallas_call(
        _tc_layer1_body,
        grid=(NBLKF,),
        in_specs=[xa, xb, _full((2 * DIN, 8 * H)), _full((1, 8 * H))],
        out_specs=[_frows(), _frows()],
        out_shape=[jax.ShapeDtypeStruct((FH, 2 * H), jnp.float32),
                   jax.ShapeDtypeStruct((FH, 2 * H), jnp.float32)],
        compiler_params=pltpu.CompilerParams(
            dimension_semantics=("parallel",)),
    )(xpad, xpad, wall, ball)


def _tc_mid(base, g, bc, wall, ball):
    gspec = pl.BlockSpec((NC, BRF, 2 * H), lambda i: (0, i, 0))
    return pl.pallas_call(
        _tc_mid_body,
        grid=(NBLKF,),
        in_specs=[_frows(), gspec, _full((1, 2 * H)),
                  _full((2 * H, 8 * H)), _full((1, 8 * H))],
        out_specs=[_frows(), _frows()],
        out_shape=[jax.ShapeDtypeStruct((FH, 2 * H), jnp.float32),
                   jax.ShapeDtypeStruct((FH, 2 * H), jnp.float32)],
        compiler_params=pltpu.CompilerParams(
            dimension_semantics=("parallel",)),
    )(base, g, bc, wall, ball)


def _tc_final(base, g, bc, onehot, wf1, bf1, wf2, bf2, ncls):
    gspec = pl.BlockSpec((NC, BRF, 2 * H), lambda i: (0, i, 0))
    oht = pl.BlockSpec((BRF, 1), lambda i: (i, 0))
    ohb = pl.BlockSpec((BRF, 1), lambda i: (i + NBLKF, 0))
    return pl.pallas_call(
        _tc_final_body,
        grid=(NBLKF,),
        in_specs=[_frows(), gspec, _full((1, 2 * H)), oht, ohb,
                  _full((H, 10)), _full((1, 10)),
                  _full((10, ncls)), _full((1, ncls))],
        out_specs=_full((NG, ncls)),
        out_shape=jax.ShapeDtypeStruct((NG, ncls), jnp.float32),
        scratch_shapes=[pltpu.VMEM((NG, H), jnp.float32),
                        pltpu.VMEM((NG, H), jnp.float32)],
        compiler_params=pltpu.CompilerParams(
            dimension_semantics=("arbitrary",)),
    )(base, g, bc, onehot, onehot, wf1, bf1, wf2, bf2)


def _fold_w(w):
    z = jnp.zeros_like(w)
    return jnp.concatenate([jnp.concatenate([w, z], axis=1),
                            jnp.concatenate([z, w], axis=1)], axis=0)


def _fold_b(b):
    return jnp.concatenate([b, b]).reshape(1, 2 * H)


def kernel(x, edge_index, batch,
           W_fc11, b_fc11, W_fc12, b_fc12, W_fc13, b_fc13, W_conv1, b_conv1,
           W_fc21, b_fc21, W_fc22, b_fc22, W_fc23, b_fc23, W_conv2, b_conv2,
           W_fc31, b_fc31, W_fc32, b_fc32, W_fc33, b_fc33, W_conv3, b_conv3,
           W_fc1, b_fc1, W_fc2, b_fc2):
    ncls = W_fc2.shape[1]

    xpad = jnp.concatenate(
        [x, jnp.zeros((NPAD - N, DIN), jnp.float32)], axis=0)

    e4 = edge_index.reshape(2, NW, NCH, CH)
    edges4 = jnp.where(e4 < FH, 2 * e4, 2 * (e4 - FH) + 1).astype(jnp.int32)
    zeros = jnp.zeros((ROWS_PT, H), jnp.float32)

    bpad = jnp.concatenate(
        [batch, jnp.full((NPAD - N,), -1, batch.dtype)]
    ).astype(jnp.int32).reshape(NPAD, 1)

    def segsum(c_f):
        out = _sc_segsum(c_f.reshape(NPAD, H), edges4, zeros)
        return out.reshape(NC, FH, 2 * H)

    def wball(w1, b1, w2, b2, w3, b3, wc):
        wall = jnp.concatenate(
            [_fold_w(w1), _fold_w(w2), _fold_w(w3), _fold_w(wc)], axis=1)
        ball = jnp.concatenate(
            [_fold_b(b1), _fold_b(b2), _fold_b(b3),
             jnp.zeros((1, 2 * H), jnp.float32)], axis=1)
        return wall, ball

    base1, c1 = _tc_layer1(
        xpad, *wball(W_fc11, b_fc11, W_fc12, b_fc12, W_fc13, b_fc13,
                     W_conv1))
    g1 = segsum(c1)
    base2, c2 = _tc_mid(
        base1, g1, _fold_b(b_conv1),
        *wball(W_fc21, b_fc21, W_fc22, b_fc22, W_fc23, b_fc23, W_conv2))
    g2 = segsum(c2)
    base3, c3 = _tc_mid(
        base2, g2, _fold_b(b_conv2),
        *wball(W_fc31, b_fc31, W_fc32, b_fc32, W_fc33, b_fc33, W_conv3))
    g3 = segsum(c3)
    return _tc_final(base3, g3, _fold_b(b_conv3), bpad,
                     W_fc1, b_fc1.reshape(1, 10), W_fc2,
                     b_fc2.reshape(1, ncls), ncls)


# speedup vs baseline: 1.1573x; 1.0127x over previous
import functools

import jax
import jax.numpy as jnp
from jax import lax
from jax.experimental import pallas as pl
from jax.experimental.pallas import tpu as pltpu
from jax.experimental.pallas import tpu_sc as plsc

N = 10000
E = 320000
DIN = 128
H = 64
NG = 16

NC = 2
NS = 16
NW = NC * NS
EPW = E // NW
CH = 80
NCH = EPW // CH
ROWS_PT = 632
NPAD = NS * ROWS_PT
SROWS = 632
FH = NPAD // 2
BRF = 2528
NBLKF = FH // BRF


def _sc_segsum_body(vals_hbm, edges_hbm, z_hbm, out_hbm,
                    sidx, didx, rows_a, rows_b, table, acc, sem_a, sem_b):
    cid = lax.axis_index("c")
    sid = lax.axis_index("s")
    wid = cid * NS + sid

    pltpu.sync_copy(edges_hbm.at[0, wid], sidx)
    pltpu.sync_copy(edges_hbm.at[1, wid], didx)
    pltpu.sync_copy(vals_hbm.at[pl.ds(sid * SROWS, SROWS)],
                    table.at[pl.ds(sid * SROWS, SROWS)])
    pltpu.sync_copy(z_hbm, acc.at[pl.ds(sid * ROWS_PT, ROWS_PT)])

    def gather(j, buf, sem):
        pltpu.async_copy(table.at[sidx.at[j]], buf, sem)

    def wait(buf, sem):
        pltpu.make_async_copy(vals_hbm.at[pl.ds(0, CH)], buf, sem).wait()

    def scatter(j, buf):
        pltpu.sync_copy(buf, acc.at[didx.at[j]], add=True)

    plsc.subcore_barrier()
    gather(0, rows_a, sem_a)

    def pair(jp, carry):
        a = 2 * jp
        gather(a + 1, rows_b, sem_b)
        wait(rows_a, sem_a)
        scatter(a, rows_a)
        gather(a + 2, rows_a, sem_a)
        wait(rows_b, sem_b)
        scatter(a + 1, rows_b)
        return carry

    lax.fori_loop(0, (NCH - 1) // 2, pair, 0)
    wait(rows_a, sem_a)
    scatter(NCH - 1, rows_a)

    plsc.subcore_barrier()
    pltpu.sync_copy(acc.at[pl.ds(sid * ROWS_PT, ROWS_PT)],
                    out_hbm.at[pl.ds(cid * NPAD + sid * ROWS_PT, ROWS_PT)])


_sc_mesh = plsc.VectorSubcoreMesh(core_axis_name="c", subcore_axis_name="s",
                                  num_cores=NC, num_subcores=NS)

_sc_segsum = functools.partial(
    pl.kernel,
    out_type=jax.ShapeDtypeStruct((NC * NPAD, H), jnp.float32),
    mesh=_sc_mesh,
    scratch_types=[
        pltpu.VMEM((NCH, CH), jnp.int32),
        pltpu.VMEM((NCH, CH), jnp.int32),
        pltpu.VMEM((CH, H), jnp.float32),
        pltpu.VMEM((CH, H), jnp.float32),
        pltpu.VMEM_SHARED((NPAD, H), jnp.float32),
        pltpu.VMEM_SHARED((NPAD, H), jnp.float32),
        pltpu.SemaphoreType.DMA,
        pltpu.SemaphoreType.DMA,
    ],
    compiler_params=pltpu.CompilerParams(use_tc_tiling_on_sc=False),
)(_sc_segsum_body)


def _dot(a, b):
    return jnp.dot(a, b, preferred_element_type=jnp.float32)


def _tc_layer1_body(xa_ref, xb_ref, wall, ball, base_ref, c_ref):
    xcat = jnp.concatenate([xa_ref[...], xb_ref[...]], axis=1)
    p = _dot(xcat, wall[...]) + ball[...]
    F = 2 * H
    base_ref[...] = p[:, :F] + p[:, F:2 * F] * p[:, 2 * F:3 * F]
    c_ref[...] = p[:, 3 * F:]


def _tc_mid_body(base_ref, g_ref, bc, wall, ball, baseo_ref, co_ref):
    h = jnp.maximum(base_ref[...] + g_ref[0] + g_ref[1] + bc[...], 0.0)
    p = _dot(h, wall[...]) + ball[...]
    F = 2 * H
    baseo_ref[...] = p[:, :F] + p[:, F:2 * F] * p[:, 2 * F:3 * F]
    co_ref[...] = p[:, 3 * F:]


def _tc_final_body(base_ref, g_ref, bc, oht_ref, ohb_ref,
                   wf1, bf1, wf2, bf2, out_ref, acc, cnt):
    i = pl.program_id(0)

    @pl.when(i == 0)
    def _():
        acc[...] = jnp.zeros_like(acc)
        cnt[...] = jnp.zeros_like(cnt)

    h = jnp.maximum(base_ref[...] + g_ref[0] + g_ref[1] + bc[...], 0.0)
    seg = lax.broadcasted_iota(jnp.int32, (1, NG), 1)
    oht = (oht_ref[...] == seg).astype(jnp.float32)
    ohb = (ohb_ref[...] == seg).astype(jnp.float32)

    def _dgt(oh, v):
        return lax.dot_general(oh, v, (((0,), (0,)), ((), ())),
                               preferred_element_type=jnp.float32)

    ones = jnp.ones((BRF, H), jnp.float32)
    acc[...] += _dgt(oht, h[:, :H]) + _dgt(ohb, h[:, H:])
    cnt[...] += _dgt(oht, ones) + _dgt(ohb, ones)

    @pl.when(i == NBLKF - 1)
    def _():
        pooled = acc[...] / jnp.maximum(cnt[...], 1.0)
        t = _dot(pooled, wf1[...]) + bf1[...]
        out_ref[...] = _dot(t, wf2[...]) + bf2[...]


def _full(shape):
    return pl.BlockSpec(shape, lambda i: tuple(0 for _ in shape))


def _frows():
    return pl.BlockSpec((BRF, 2 * H), lambda i: (i, 0))


def _tc_layer1(xpad, wall, ball):
    xa = pl.BlockSpec((BRF, DIN), lambda i: (i, 0))
    xb = pl.BlockSpec((BRF, DIN), lambda i: (i + NBLKF, 0))
    return pl.pallas_call(
        _tc_layer1_body,
        grid=(NBLKF,),
        in_specs=[xa, xb, _full((2 * DIN, 8 * H)), _full((1, 8 * H))],
        out_specs=[_frows(), _frows()],
        out_shape=[jax.ShapeDtypeStruct((FH, 2 * H), jnp.float32),
                   jax.ShapeDtypeStruct((FH, 2 * H), jnp.float32)],
        compiler_params=pltpu.CompilerParams(
            dimension_semantics=("parallel",)),
    )(xpad, xpad, wall, ball)


def _tc_mid(base, g, bc, wall, ball):
    gspec = pl.BlockSpec((NC, BRF, 2 * H), lambda i: (0, i, 0))
    return pl.pallas_call(
        _tc_mid_body,
        grid=(NBLKF,),
        in_specs=[_frows(), gspec, _full((1, 2 * H)),
                  _full((2 * H, 8 * H)), _full((1, 8 * H))],
        out_specs=[_frows(), _frows()],
        out_shape=[jax.ShapeDtypeStruct((FH, 2 * H), jnp.float32),
                   jax.ShapeDtypeStruct((FH, 2 * H), jnp.float32)],
        compiler_params=pltpu.CompilerParams(
            dimension_semantics=("parallel",)),
    )(base, g, bc, wall, ball)


def _tc_final(base, g, bc, onehot, wf1, bf1, wf2, bf2, ncls):
    gspec = pl.BlockSpec((NC, BRF, 2 * H), lambda i: (0, i, 0))
    oht = pl.BlockSpec((BRF, 1), lambda i: (i, 0))
    ohb = pl.BlockSpec((BRF, 1), lambda i: (i + NBLKF, 0))
    return pl.pallas_call(
        _tc_final_body,
        grid=(NBLKF,),
        in_specs=[_frows(), gspec, _full((1, 2 * H)), oht, ohb,
                  _full((H, 10)), _full((1, 10)),
                  _full((10, ncls)), _full((1, ncls))],
        out_specs=_full((NG, ncls)),
        out_shape=jax.ShapeDtypeStruct((NG, ncls), jnp.float32),
        scratch_shapes=[pltpu.VMEM((NG, H), jnp.float32),
                        pltpu.VMEM((NG, H), jnp.float32)],
        compiler_params=pltpu.CompilerParams(
            dimension_semantics=("arbitrary",)),
    )(base, g, bc, onehot, onehot, wf1, bf1, wf2, bf2)


def _fold_w(w):
    z = jnp.zeros_like(w)
    return jnp.concatenate([jnp.concatenate([w, z], axis=1),
                            jnp.concatenate([z, w], axis=1)], axis=0)


def _fold_b(b):
    return jnp.concatenate([b, b]).reshape(1, 2 * H)


def kernel(x, edge_index, batch,
           W_fc11, b_fc11, W_fc12, b_fc12, W_fc13, b_fc13, W_conv1, b_conv1,
           W_fc21, b_fc21, W_fc22, b_fc22, W_fc23, b_fc23, W_conv2, b_conv2,
           W_fc31, b_fc31, W_fc32, b_fc32, W_fc33, b_fc33, W_conv3, b_conv3,
           W_fc1, b_fc1, W_fc2, b_fc2):
    ncls = W_fc2.shape[1]

    xpad = jnp.concatenate(
        [x, jnp.zeros((NPAD - N, DIN), jnp.float32)], axis=0)

    e4 = edge_index.reshape(2, NW, NCH, CH)
    edges4 = jnp.where(e4 < FH, 2 * e4, 2 * (e4 - FH) + 1).astype(jnp.int32)
    zeros = jnp.zeros((ROWS_PT, H), jnp.float32)

    bpad = jnp.concatenate(
        [batch, jnp.full((NPAD - N,), -1, batch.dtype)]
    ).astype(jnp.int32).reshape(NPAD, 1)

    def segsum(c_f):
        out = _sc_segsum(c_f.reshape(NPAD, H), edges4, zeros)
        return out.reshape(NC, FH, 2 * H)

    def wball(w1, b1, w2, b2, w3, b3, wc):
        wall = jnp.concatenate(
            [_fold_w(w1), _fold_w(w2), _fold_w(w3), _fold_w(wc)], axis=1)
        ball = jnp.concatenate(
            [_fold_b(b1), _fold_b(b2), _fold_b(b3),
             jnp.zeros((1, 2 * H), jnp.float32)], axis=1)
        return wall, ball

    base1, c1 = _tc_layer1(
        xpad, *wball(W_fc11, b_fc11, W_fc12, b_fc12, W_fc13, b_fc13,
                     W_conv1))
    g1 = segsum(c1)
    base2, c2 = _tc_mid(
        base1, g1, _fold_b(b_conv1),
        *wball(W_fc21, b_fc21, W_fc22, b_fc22, W_fc23, b_fc23, W_conv2))
    g2 = segsum(c2)
    base3, c3 = _tc_mid(
        base2, g2, _fold_b(b_conv2),
        *wball(W_fc31, b_fc31, W_fc32, b_fc32, W_fc33, b_fc33, W_conv3))
    g3 = segsum(c3)
    return _tc_final(base3, g3, _fold_b(b_conv3), bpad,
                     W_fc1, b_fc1.reshape(1, 10), W_fc2,
                     b_fc2.reshape(1, ncls), ncls)


# speedup vs baseline: 1.1785x; 1.0183x over previous
import functools

import jax
import jax.numpy as jnp
from jax import lax
from jax.experimental import pallas as pl
from jax.experimental.pallas import tpu as pltpu
from jax.experimental.pallas import tpu_sc as plsc

N = 10000
E = 320000
DIN = 128
H = 64
NG = 16

NC = 2
NS = 16
NW = NC * NS
EPW = E // NW
CH = 80
NCH = EPW // CH
ROWS_PT = 632
NPAD = NS * ROWS_PT
SROWS = 632
FH = NPAD // 2
BRF = 2528
NBLKF = FH // BRF


def _sc_segsum_body(vals_hbm, edges_hbm, z_hbm, out_hbm,
                    sidx, didx, rows_a, rows_b, table, acc,
                    sem_a, sem_b, sem_z):
    cid = lax.axis_index("c")
    sid = lax.axis_index("s")
    wid = cid * NS + sid

    pltpu.async_copy(edges_hbm.at[0, wid], sidx, sem_a)
    pltpu.async_copy(edges_hbm.at[1, wid], didx, sem_b)
    pltpu.async_copy(z_hbm, acc.at[pl.ds(sid * ROWS_PT, ROWS_PT)], sem_z)
    pltpu.sync_copy(vals_hbm.at[pl.ds(sid * SROWS, SROWS)],
                    table.at[pl.ds(sid * SROWS, SROWS)])
    pltpu.make_async_copy(edges_hbm.at[0, wid], sidx, sem_a).wait()
    pltpu.make_async_copy(edges_hbm.at[1, wid], didx, sem_b).wait()
    pltpu.make_async_copy(
        z_hbm, acc.at[pl.ds(sid * ROWS_PT, ROWS_PT)], sem_z).wait()

    def gather(j, buf, sem):
        pltpu.async_copy(table.at[sidx.at[j]], buf, sem)

    def wait(buf, sem):
        pltpu.make_async_copy(vals_hbm.at[pl.ds(0, CH)], buf, sem).wait()

    def scatter(j, buf):
        pltpu.sync_copy(buf, acc.at[didx.at[j]], add=True)

    plsc.subcore_barrier()
    gather(0, rows_a, sem_a)

    def pair(jp, carry):
        a = 2 * jp
        gather(a + 1, rows_b, sem_b)
        wait(rows_a, sem_a)
        scatter(a, rows_a)
        gather(a + 2, rows_a, sem_a)
        wait(rows_b, sem_b)
        scatter(a + 1, rows_b)
        return carry

    lax.fori_loop(0, (NCH - 1) // 2, pair, 0)
    wait(rows_a, sem_a)
    scatter(NCH - 1, rows_a)

    plsc.subcore_barrier()
    pltpu.sync_copy(acc.at[pl.ds(sid * ROWS_PT, ROWS_PT)],
                    out_hbm.at[pl.ds(cid * NPAD + sid * ROWS_PT, ROWS_PT)])


_sc_mesh = plsc.VectorSubcoreMesh(core_axis_name="c", subcore_axis_name="s",
                                  num_cores=NC, num_subcores=NS)

_sc_segsum = functools.partial(
    pl.kernel,
    out_type=jax.ShapeDtypeStruct((NC * NPAD, H), jnp.float32),
    mesh=_sc_mesh,
    scratch_types=[
        pltpu.VMEM((NCH, CH), jnp.int32),
        pltpu.VMEM((NCH, CH), jnp.int32),
        pltpu.VMEM((CH, H), jnp.float32),
        pltpu.VMEM((CH, H), jnp.float32),
        pltpu.VMEM_SHARED((NPAD, H), jnp.float32),
        pltpu.VMEM_SHARED((NPAD, H), jnp.float32),
        pltpu.SemaphoreType.DMA,
        pltpu.SemaphoreType.DMA,
        pltpu.SemaphoreType.DMA,
    ],
    compiler_params=pltpu.CompilerParams(use_tc_tiling_on_sc=False),
)(_sc_segsum_body)


def _dot(a, b):
    return jnp.dot(a, b, preferred_element_type=jnp.float32)


def _tc_layer1_body(xa_ref, xb_ref, wall, ball, base_ref, c_ref):
    xcat = jnp.concatenate([xa_ref[...], xb_ref[...]], axis=1)
    p = _dot(xcat, wall[...]) + ball[...]
    F = 2 * H
    base_ref[...] = p[:, :F] + p[:, F:2 * F] * p[:, 2 * F:3 * F]
    c_ref[...] = p[:, 3 * F:]


def _tc_mid_body(base_ref, g_ref, bc, wall, ball, baseo_ref, co_ref):
    h = jnp.maximum(base_ref[...] + g_ref[0] + g_ref[1] + bc[...], 0.0)
    p = _dot(h, wall[...]) + ball[...]
    F = 2 * H
    baseo_ref[...] = p[:, :F] + p[:, F:2 * F] * p[:, 2 * F:3 * F]
    co_ref[...] = p[:, 3 * F:]


def _tc_final_body(base_ref, g_ref, bc, oht_ref, ohb_ref,
                   wf1, bf1, wf2, bf2, out_ref, acc, cnt):
    i = pl.program_id(0)

    @pl.when(i == 0)
    def _():
        acc[...] = jnp.zeros_like(acc)
        cnt[...] = jnp.zeros_like(cnt)

    h = jnp.maximum(base_ref[...] + g_ref[0] + g_ref[1] + bc[...], 0.0)
    seg = lax.broadcasted_iota(jnp.int32, (1, NG), 1)
    oht = (oht_ref[...] == seg).astype(jnp.float32)
    ohb = (ohb_ref[...] == seg).astype(jnp.float32)

    def _dgt(oh, v):
        return lax.dot_general(oh, v, (((0,), (0,)), ((), ())),
                               preferred_element_type=jnp.float32)

    ones = jnp.ones((BRF, H), jnp.float32)
    acc[...] += _dgt(oht, h[:, :H]) + _dgt(ohb, h[:, H:])
    cnt[...] += _dgt(oht, ones) + _dgt(ohb, ones)

    @pl.when(i == NBLKF - 1)
    def _():
        pooled = acc[...] / jnp.maximum(cnt[...], 1.0)
        t = _dot(pooled, wf1[...]) + bf1[...]
        out_ref[...] = _dot(t, wf2[...]) + bf2[...]


def _full(shape):
    return pl.BlockSpec(shape, lambda i: tuple(0 for _ in shape))


def _frows():
    return pl.BlockSpec((BRF, 2 * H), lambda i: (i, 0))


def _tc_layer1(xpad, wall, ball):
    xa = pl.BlockSpec((BRF, DIN), lambda i: (i, 0))
    xb = pl.BlockSpec((BRF, DIN), lambda i: (i + NBLKF, 0))
    return pl.pallas_call(
        _tc_layer1_body,
        grid=(NBLKF,),
        in_specs=[xa, xb, _full((2 * DIN, 8 * H)), _full((1, 8 * H))],
        out_specs=[_frows(), _frows()],
        out_shape=[jax.ShapeDtypeStruct((FH, 2 * H), jnp.float32),
                   jax.ShapeDtypeStruct((FH, 2 * H), jnp.float32)],
        compiler_params=pltpu.CompilerParams(
            dimension_semantics=("parallel",)),
    )(xpad, xpad, wall, ball)


def _tc_mid(base, g, bc, wall, ball):
    gspec = pl.BlockSpec((NC, BRF, 2 * H), lambda i: (0, i, 0))
    return pl.pallas_call(
        _tc_mid_body,
        grid=(NBLKF,),
        in_specs=[_frows(), gspec, _full((1, 2 * H)),
                  _full((2 * H, 8 * H)), _full((1, 8 * H))],
        out_specs=[_frows(), _frows()],
        out_shape=[jax.ShapeDtypeStruct((FH, 2 * H), jnp.float32),
                   jax.ShapeDtypeStruct((FH, 2 * H), jnp.float32)],
        compiler_params=pltpu.CompilerParams(
            dimension_semantics=("parallel",)),
    )(base, g, bc, wall, ball)


def _tc_final(base, g, bc, onehot, wf1, bf1, wf2, bf2, ncls):
    gspec = pl.BlockSpec((NC, BRF, 2 * H), lambda i: (0, i, 0))
    oht = pl.BlockSpec((BRF, 1), lambda i: (i, 0))
    ohb = pl.BlockSpec((BRF, 1), lambda i: (i + NBLKF, 0))
    return pl.pallas_call(
        _tc_final_body,
        grid=(NBLKF,),
        in_specs=[_frows(), gspec, _full((1, 2 * H)), oht, ohb,
                  _full((H, 10)), _full((1, 10)),
                  _full((10, ncls)), _full((1, ncls))],
        out_specs=_full((NG, ncls)),
        out_shape=jax.ShapeDtypeStruct((NG, ncls), jnp.float32),
        scratch_shapes=[pltpu.VMEM((NG, H), jnp.float32),
                        pltpu.VMEM((NG, H), jnp.float32)],
        compiler_params=pltpu.CompilerParams(
            dimension_semantics=("arbitrary",)),
    )(base, g, bc, onehot, onehot, wf1, bf1, wf2, bf2)


def _fold_w(w):
    z = jnp.zeros_like(w)
    return jnp.concatenate([jnp.concatenate([w, z], axis=1),
                            jnp.concatenate([z, w], axis=1)], axis=0)


def _fold_b(b):
    return jnp.concatenate([b, b]).reshape(1, 2 * H)


def kernel(x, edge_index, batch,
           W_fc11, b_fc11, W_fc12, b_fc12, W_fc13, b_fc13, W_conv1, b_conv1,
           W_fc21, b_fc21, W_fc22, b_fc22, W_fc23, b_fc23, W_conv2, b_conv2,
           W_fc31, b_fc31, W_fc32, b_fc32, W_fc33, b_fc33, W_conv3, b_conv3,
           W_fc1, b_fc1, W_fc2, b_fc2):
    ncls = W_fc2.shape[1]

    xpad = jnp.concatenate(
        [x, jnp.zeros((NPAD - N, DIN), jnp.float32)], axis=0)

    e4 = edge_index.reshape(2, NW, NCH, CH)
    edges4 = jnp.where(e4 < FH, 2 * e4, 2 * (e4 - FH) + 1).astype(jnp.int32)
    zeros = jnp.zeros((ROWS_PT, H), jnp.float32)

    bpad = jnp.concatenate(
        [batch, jnp.full((NPAD - N,), -1, batch.dtype)]
    ).astype(jnp.int32).reshape(NPAD, 1)

    def segsum(c_f):
        out = _sc_segsum(c_f.reshape(NPAD, H), edges4, zeros)
        return out.reshape(NC, FH, 2 * H)

    def wball(w1, b1, w2, b2, w3, b3, wc):
        wall = jnp.concatenate(
            [_fold_w(w1), _fold_w(w2), _fold_w(w3), _fold_w(wc)], axis=1)
        ball = jnp.concatenate(
            [_fold_b(b1), _fold_b(b2), _fold_b(b3),
             jnp.zeros((1, 2 * H), jnp.float32)], axis=1)
        return wall, ball

    base1, c1 = _tc_layer1(
        xpad, *wball(W_fc11, b_fc11, W_fc12, b_fc12, W_fc13, b_fc13,
                     W_conv1))
    g1 = segsum(c1)
    base2, c2 = _tc_mid(
        base1, g1, _fold_b(b_conv1),
        *wball(W_fc21, b_fc21, W_fc22, b_fc22, W_fc23, b_fc23, W_conv2))
    g2 = segsum(c2)
    base3, c3 = _tc_mid(
        base2, g2, _fold_b(b_conv2),
        *wball(W_fc31, b_fc31, W_fc32, b_fc32, W_fc33, b_fc33, W_conv3))
    g3 = segsum(c3)
    return _tc_final(base3, g3, _fold_b(b_conv3), bpad,
                     W_fc1, b_fc1.reshape(1, 10), W_fc2,
                     b_fc2.reshape(1, ncls), ncls)
